# baseline (speedup 1.0000x reference)
import functools
import math

import jax
import jax.numpy as jnp
from jax import lax
from jax.experimental import pallas as pl
from jax.experimental.pallas import tpu as pltpu
from jax.experimental.pallas import tpu_sc as plsc

DIM = 64
SCALE = math.sqrt(DIM)

_info = plsc.get_sparse_core_info()
_NC = _info.num_cores
_NS = _info.num_subcores
_NW = _NC * _NS


@functools.lru_cache(maxsize=None)
def _make_emb(B: int, C: int):
    assert B % _NW == 0
    b_per_w = B // _NW
    assert b_per_w % C == 0
    nch = b_per_w // C
    mesh = plsc.VectorSubcoreMesh(core_axis_name="c", subcore_axis_name="s")

    @functools.partial(
        pl.kernel,
        out_type=jax.ShapeDtypeStruct((B, DIM), jnp.float32),
        mesh=mesh,
        scratch_types=[
            pltpu.VMEM((C,), jnp.int32),
            pltpu.VMEM((C, DIM), jnp.float32),
            pltpu.SemaphoreType.DMA,
        ],
        compiler_params=pltpu.CompilerParams(use_tc_tiling_on_sc=False),
    )
    def emb(x_hbm, lut_hbm, out_hbm, idx_v, rows_v, gsem):
        wid = lax.axis_index("s") * _NC + lax.axis_index("c")
        base = wid * b_per_w

        def chunk(g, carry):
            row0 = base + g * C
            pltpu.sync_copy(x_hbm.at[pl.ds(row0, C)], idx_v)
            pltpu.async_copy(lut_hbm.at[idx_v], rows_v, gsem).wait()

            def srow(r, c2):
                for q in range(DIM // 16):
                    sl = pl.ds(q * 16, 16)
                    rows_v[r, sl] = rows_v[r, sl] * SCALE
                return c2

            lax.fori_loop(0, C, srow, 0, unroll=4)
            pltpu.sync_copy(rows_v, out_hbm.at[pl.ds(row0, C)])
            return carry

        lax.fori_loop(0, nch, chunk, 0)

    return emb


def kernel(x, lut):
    B = x.size
    out = _make_emb(B, 800)(x.reshape(-1), lut)
    return out.reshape(*x.shape, DIM)


# speedup vs baseline: 1.0711x; 1.0711x over previous
import functools
import math

import jax
import jax.numpy as jnp
from jax import lax
from jax.experimental import pallas as pl
from jax.experimental.pallas import tpu as pltpu
from jax.experimental.pallas import tpu_sc as plsc

DIM = 64
SCALE = math.sqrt(DIM)
NBUF = 2

_info = plsc.get_sparse_core_info()
_NC = _info.num_cores
_NS = _info.num_subcores
_NW = _NC * _NS


@functools.lru_cache(maxsize=None)
def _make_emb(B: int, C: int):
    assert B % _NW == 0
    b_per_w = B // _NW
    assert b_per_w % C == 0 and C % 8 == 0
    nch = b_per_w // C
    assert nch % NBUF == 0 and nch > NBUF
    mesh = plsc.VectorSubcoreMesh(core_axis_name="c", subcore_axis_name="s")

    @functools.partial(
        pl.kernel,
        out_type=jax.ShapeDtypeStruct((B, DIM), jnp.float32),
        mesh=mesh,
        scratch_types=[
            pltpu.VMEM((nch, C), jnp.int32),
            [pltpu.VMEM((C, DIM), jnp.float32) for _ in range(NBUF)],
            [pltpu.VMEM((C, DIM), jnp.float32) for _ in range(NBUF)],
            [pltpu.SemaphoreType.DMA for _ in range(NBUF)],
            [pltpu.SemaphoreType.DMA for _ in range(NBUF)],
        ],
        compiler_params=pltpu.CompilerParams(use_tc_tiling_on_sc=False),
    )
    def emb(x_hbm, lut_hbm, out_hbm, idx_v, rows_in, rows_out, gsem, osem):
        wid = lax.axis_index("s") * _NC + lax.axis_index("c")
        base = wid * b_per_w
        pltpu.sync_copy(x_hbm.at[pl.ds(wid * nch, nch)], idx_v)

        def gather(g, b):
            return pltpu.make_async_copy(
                lut_hbm.at[idx_v.at[g]], rows_in[b], gsem[b])

        def scatter(g, b):
            return pltpu.make_async_copy(
                rows_out[b], out_hbm.at[pl.ds(base + g * C, C)], osem[b])

        for b in range(NBUF):
            gather(jnp.int32(b), b).start()

        def outer(i, carry):
            go = i * NBUF
            for b in range(NBUF):
                g = go + b
                gather(g, b).wait()

                @pl.when(g >= NBUF)
                def _():
                    scatter(g - NBUF, b).wait()

                @plsc.parallel_loop(0, C, unroll=8)
                def _(r):
                    for q in range(DIM // 16):
                        sl = pl.ds(q * 16, 16)
                        rows_out[b][r, sl] = rows_in[b][r, sl] * SCALE

                @pl.when(g + NBUF < nch)
                def _():
                    gather(g + NBUF, b).start()

                scatter(g, b).start()
            return carry

        lax.fori_loop(0, nch // NBUF, outer, 0)
        for b in range(NBUF):
            scatter(jnp.int32(nch - NBUF + b), b).wait()

    return emb


def kernel(x, lut):
    B = x.size
    C = 400
    out = _make_emb(B, C)(x.reshape(B // C, C), lut)
    return out.reshape(*x.shape, DIM)
